# baseline (speedup 1.0000x reference)
import jax
import jax.numpy as jnp
from jax import lax
from jax.experimental import pallas as pl
from jax.experimental.pallas import tpu as pltpu
from jax.experimental.pallas import tpu_sc as plsc

VOCAB = 1000000
EMBED_DIM = 32
BATCH = 16384

_INFO = plsc.get_sparse_core_info()
_NC = _INFO.num_cores
_NS = _INFO.num_subcores
_NW = _NC * _NS
_B_PER_W = BATCH // _NW
_QROWS = VOCAB // 4


def _triplet_gather(a_hbm, p_hbm, n_hbm, wq_hbm,
                    out_a, out_p, out_n,
                    idx_v, q_v, rows_v, slab, sem):
    wid = lax.axis_index("s") * _NC + lax.axis_index("c")
    base = wid * _B_PER_W
    iota = lax.iota(jnp.int32, 16)

    def do_lookup(idx_hbm, out_hbm):
        pltpu.sync_copy(idx_hbm.at[pl.ds(base, _B_PER_W)], idx_v)

        def prep(c, _):
            chunk = idx_v[pl.ds(c * 16, 16)]
            q_v[pl.ds(c * 16, 16)] = lax.shift_right_logical(chunk, 2)
            return 0

        lax.fori_loop(0, _B_PER_W // 16, prep, 0)
        pltpu.async_copy(wq_hbm.at[q_v], rows_v, sem).wait()

        def extract(c, _):
            chunk = idx_v[pl.ds(c * 16, 16)]
            lane0 = lax.shift_left(lax.bitwise_and(chunk, jnp.int32(3)), 5)
            pos = iota + c * 16
            for d in range(EMBED_DIM):
                vals = plsc.load_gather(rows_v, [pos, lane0 + d])
                slab[d, pl.ds(c * 16, 16)] = vals
            return 0

        lax.fori_loop(0, _B_PER_W // 16, extract, 0)
        pltpu.sync_copy(slab, out_hbm.at[:, pl.ds(base, _B_PER_W)])

    do_lookup(a_hbm, out_a)
    do_lookup(p_hbm, out_p)
    do_lookup(n_hbm, out_n)


@jax.jit
def kernel(anchor, positive, negative, W):
    wq = jnp.reshape(W, (_QROWS, 128))
    mesh = plsc.VectorSubcoreMesh(core_axis_name="c", subcore_axis_name="s")
    out_t = jax.ShapeDtypeStruct((EMBED_DIM, BATCH), jnp.float32)
    f = pl.kernel(
        _triplet_gather,
        mesh=mesh,
        out_type=(out_t, out_t, out_t),
        scratch_types=[
            pltpu.VMEM((_B_PER_W,), jnp.int32),
            pltpu.VMEM((_B_PER_W,), jnp.int32),
            pltpu.VMEM((_B_PER_W, 128), jnp.float32),
            pltpu.VMEM((EMBED_DIM, _B_PER_W), jnp.float32),
            pltpu.SemaphoreType.DMA,
        ],
        compiler_params=pltpu.CompilerParams(needs_layout_passes=False),
    )
    oa, op_, on = f(anchor, positive, negative, wq)
    return (oa.T, op_.T, on.T)


# speedup vs baseline: 1.0214x; 1.0214x over previous
import jax
import jax.numpy as jnp
from jax import lax
from jax.experimental import pallas as pl
from jax.experimental.pallas import tpu as pltpu
from jax.experimental.pallas import tpu_sc as plsc

VOCAB = 1000000
EMBED_DIM = 32
BATCH = 16384

_INFO = plsc.get_sparse_core_info()
_NC = _INFO.num_cores
_NS = _INFO.num_subcores
_NW = _NC * _NS
_B_PER_W = BATCH // _NW


def _triplet_gather(a_hbm, p_hbm, n_hbm, table_hbm,
                    out_a, out_p, out_n,
                    ia_v, ip_v, in_v, ra_v, rp_v, rn_v,
                    sem_a, sem_p, sem_n):
    wid = lax.axis_index("s") * _NC + lax.axis_index("c")
    base = wid * _B_PER_W
    sl = pl.ds(base, _B_PER_W)
    pltpu.sync_copy(a_hbm.at[sl], ia_v)
    pltpu.sync_copy(p_hbm.at[sl], ip_v)
    pltpu.sync_copy(n_hbm.at[sl], in_v)
    ca = pltpu.async_copy(table_hbm.at[ia_v], ra_v, sem_a)
    cp = pltpu.async_copy(table_hbm.at[ip_v], rp_v, sem_p)
    cn = pltpu.async_copy(table_hbm.at[in_v], rn_v, sem_n)
    ca.wait()
    pltpu.sync_copy(ra_v, out_a.at[sl])
    cp.wait()
    pltpu.sync_copy(rp_v, out_p.at[sl])
    cn.wait()
    pltpu.sync_copy(rn_v, out_n.at[sl])


@jax.jit
def kernel(anchor, positive, negative, W):
    mesh = plsc.VectorSubcoreMesh(core_axis_name="c", subcore_axis_name="s")
    out = jax.ShapeDtypeStruct((BATCH, EMBED_DIM), jnp.float32)
    f = pl.kernel(
        _triplet_gather,
        mesh=mesh,
        out_type=(out, out, out),
        scratch_types=[
            pltpu.VMEM((_B_PER_W,), jnp.int32),
            pltpu.VMEM((_B_PER_W,), jnp.int32),
            pltpu.VMEM((_B_PER_W,), jnp.int32),
            pltpu.VMEM((_B_PER_W, EMBED_DIM), jnp.float32),
            pltpu.VMEM((_B_PER_W, EMBED_DIM), jnp.float32),
            pltpu.VMEM((_B_PER_W, EMBED_DIM), jnp.float32),
            pltpu.SemaphoreType.DMA,
            pltpu.SemaphoreType.DMA,
            pltpu.SemaphoreType.DMA,
        ],
        compiler_params=pltpu.CompilerParams(use_tc_tiling_on_sc=False),
    )
    return f(anchor, positive, negative, W)


# speedup vs baseline: 1.2355x; 1.2096x over previous
import jax
import jax.numpy as jnp
from jax import lax
from jax.experimental import pallas as pl
from jax.experimental.pallas import tpu as pltpu
from jax.experimental.pallas import tpu_sc as plsc

VOCAB = 1000000
EMBED_DIM = 32
BATCH = 16384

_INFO = plsc.get_sparse_core_info()
_NC = _INFO.num_cores
_NS = _INFO.num_subcores
_NW = _NC * _NS
_B_PER_W = BATCH // _NW
_NBUF = 8


def _triplet_gather(a_hbm, p_hbm, n_hbm, w_hbm,
                    out_a, out_p, out_n,
                    idx_vmem, blocks, slab, sems):
    wid = lax.axis_index("s") * _NC + lax.axis_index("c")
    base = wid * _B_PER_W
    iota = lax.iota(jnp.int32, 16)

    def do_lookup(idx_hbm, out_hbm):
        pltpu.sync_copy(idx_hbm.at[pl.ds(base, _B_PER_W)], idx_vmem)

        def read_idx(i):
            chunk = idx_vmem[pl.ds((i // 16) * 16, 16)]
            lane = jnp.broadcast_to(lax.rem(i, 16), (16,))
            sel = jnp.where(iota == lane, chunk, jnp.int32(0))
            return lax.reduce_max(sel, (0,))

        def fetch(i, buf):
            s = read_idx(i)
            r0 = pl.multiple_of(lax.shift_left(lax.shift_right_logical(s, 3), 3), 8)
            pltpu.async_copy(
                w_hbm.at[pl.ds(r0, 8), :], blocks.at[buf], sems.at[buf]
            )

        for b in range(_NBUF):
            fetch(b, b)

        def body(g, _):
            for b in range(_NBUF):
                i = g * _NBUF + b
                pltpu.make_async_copy(
                    w_hbm.at[pl.ds(0, 8), :], blocks.at[b], sems.at[b]
                ).wait()
                s = read_idx(i)
                sub = lax.bitwise_and(s, jnp.int32(7))
                col = jnp.broadcast_to(i, (16,))
                lo = blocks.at[b][sub, pl.ds(0, 16)]
                hi = blocks.at[b][sub, pl.ds(16, 16)]
                plsc.store_scatter(slab, [iota, col], lo)
                plsc.store_scatter(slab, [iota + 16, col], hi)
                nxt = i + _NBUF

                @pl.when(nxt < _B_PER_W)
                def _():
                    fetch(nxt, b)

            return 0

        lax.fori_loop(0, _B_PER_W // _NBUF, body, 0)
        pltpu.sync_copy(slab, out_hbm.at[:, pl.ds(base, _B_PER_W)])

    do_lookup(a_hbm, out_a)
    do_lookup(p_hbm, out_p)
    do_lookup(n_hbm, out_n)


@jax.jit
def kernel(anchor, positive, negative, W):
    mesh = plsc.VectorSubcoreMesh(core_axis_name="c", subcore_axis_name="s")
    out_t = jax.ShapeDtypeStruct((EMBED_DIM, BATCH), jnp.float32)
    f = pl.kernel(
        _triplet_gather,
        mesh=mesh,
        out_type=(out_t, out_t, out_t),
        scratch_types=[
            pltpu.VMEM((_B_PER_W,), jnp.int32),
            pltpu.VMEM((_NBUF, 8, EMBED_DIM), jnp.float32),
            pltpu.VMEM((EMBED_DIM, _B_PER_W), jnp.float32),
            pltpu.SemaphoreType.DMA((_NBUF,)),
        ],
        compiler_params=pltpu.CompilerParams(needs_layout_passes=False),
    )
    oa, op_, on = f(anchor, positive, negative, W)
    return (oa.T, op_.T, on.T)


# speedup vs baseline: 1.2479x; 1.0101x over previous
import jax
import jax.numpy as jnp
from jax import lax
from jax.experimental import pallas as pl
from jax.experimental.pallas import tpu as pltpu
from jax.experimental.pallas import tpu_sc as plsc

VOCAB = 1000000
EMBED_DIM = 32
BATCH = 16384

_INFO = plsc.get_sparse_core_info()
_NC = _INFO.num_cores
_NS = _INFO.num_subcores
_NW = _NC * _NS
_B_PER_W = BATCH // _NW
_NBUF = 8


def _triplet_gather(a_hbm, p_hbm, n_hbm, w_hbm,
                    out_a, out_p, out_n,
                    idx_vmem, blocks, slab, sems):
    wid = lax.axis_index("s") * _NC + lax.axis_index("c")
    base = wid * _B_PER_W
    iota = lax.iota(jnp.int32, 16)

    def do_lookup(idx_hbm, out_hbm):
        pltpu.sync_copy(idx_hbm.at[pl.ds(base, _B_PER_W)], idx_vmem)

        def read_idx(i):
            chunk = idx_vmem[pl.ds((i // 16) * 16, 16)]
            lane = jnp.broadcast_to(lax.rem(i, 16), (16,))
            sel = jnp.where(iota == lane, chunk, jnp.int32(0))
            return lax.reduce_max(sel, (0,))

        def fetch(i, buf):
            s = read_idx(i)
            r0 = pl.multiple_of(lax.shift_left(lax.shift_right_logical(s, 3), 3), 8)
            pltpu.async_copy(
                w_hbm.at[pl.ds(r0, 8), :], blocks.at[buf], sems.at[buf]
            )

        for b in range(_NBUF):
            fetch(b, b)

        def body(g, _):
            for b in range(_NBUF):
                i = g * _NBUF + b
                pltpu.make_async_copy(
                    w_hbm.at[pl.ds(0, 8), :], blocks.at[b], sems.at[b]
                ).wait()
                s_v = plsc.load_gather(idx_vmem, [jnp.broadcast_to(i, (16,))])
                sub_v = lax.bitwise_and(s_v, jnp.int32(7))
                col = jnp.broadcast_to(i, (16,))
                lo = plsc.load_gather(blocks.at[b], [sub_v, iota])
                hi = plsc.load_gather(blocks.at[b], [sub_v, iota + 16])
                plsc.store_scatter(slab, [iota, col], lo)
                plsc.store_scatter(slab, [iota + 16, col], hi)
                nxt = i + _NBUF

                @pl.when(nxt < _B_PER_W)
                def _():
                    fetch(nxt, b)

            return 0

        lax.fori_loop(0, _B_PER_W // _NBUF, body, 0)
        pltpu.sync_copy(slab, out_hbm.at[:, pl.ds(base, _B_PER_W)])

    do_lookup(a_hbm, out_a)
    do_lookup(p_hbm, out_p)
    do_lookup(n_hbm, out_n)


@jax.jit
def kernel(anchor, positive, negative, W):
    mesh = plsc.VectorSubcoreMesh(core_axis_name="c", subcore_axis_name="s")
    out_t = jax.ShapeDtypeStruct((EMBED_DIM, BATCH), jnp.float32)
    f = pl.kernel(
        _triplet_gather,
        mesh=mesh,
        out_type=(out_t, out_t, out_t),
        scratch_types=[
            pltpu.VMEM((_B_PER_W,), jnp.int32),
            pltpu.VMEM((_NBUF, 8, EMBED_DIM), jnp.float32),
            pltpu.VMEM((EMBED_DIM, _B_PER_W), jnp.float32),
            pltpu.SemaphoreType.DMA((_NBUF,)),
        ],
        compiler_params=pltpu.CompilerParams(needs_layout_passes=False),
    )
    oa, op_, on = f(anchor, positive, negative, W)
    return (oa.T, op_.T, on.T)


# speedup vs baseline: 1.7169x; 1.3758x over previous
import jax
import jax.numpy as jnp
from jax import lax
from jax.experimental import pallas as pl
from jax.experimental.pallas import tpu as pltpu
from jax.experimental.pallas import tpu_sc as plsc

VOCAB = 1000000
EMBED_DIM = 32
BATCH = 16384

_INFO = plsc.get_sparse_core_info()
_NC = _INFO.num_cores
_NS = _INFO.num_subcores
_NW = _NC * _NS
_B_PER_W = BATCH // _NW
_NBUF = 8


def _triplet_gather(a_hbm, p_hbm, n_hbm, w_hbm,
                    out_a, out_p, out_n,
                    idx_vmem, blocks, slab, sems):
    wid = lax.axis_index("s") * _NC + lax.axis_index("c")
    base = wid * _B_PER_W
    iota = lax.iota(jnp.int32, 16)

    def do_lookup(idx_hbm, out_hbm):
        pltpu.sync_copy(idx_hbm.at[pl.ds(base, _B_PER_W)], idx_vmem)

        def read_idx(i):
            chunk = idx_vmem[pl.ds((i // 16) * 16, 16)]
            lane = jnp.broadcast_to(lax.rem(i, 16), (16,))
            sel = jnp.where(iota == lane, chunk, jnp.int32(0))
            return lax.reduce_max(sel, (0,))

        def fetch(i, buf):
            s = read_idx(i)
            t0 = lax.shift_right_logical(s, 3)
            pltpu.async_copy(
                w_hbm.at[t0], blocks.at[buf], sems.at[buf]
            )

        for b in range(_NBUF):
            fetch(b, b)

        def body(g, _):
            for b in range(_NBUF):
                i = g * _NBUF + b
                pltpu.make_async_copy(
                    w_hbm.at[0], blocks.at[b], sems.at[b]
                ).wait()
                s_v = plsc.load_gather(idx_vmem, [jnp.broadcast_to(i, (16,))])
                sub_v = lax.bitwise_and(s_v, jnp.int32(7))
                col = jnp.broadcast_to(i, (16,))
                lo = plsc.load_gather(blocks.at[b], [sub_v, iota])
                hi = plsc.load_gather(blocks.at[b], [sub_v, iota + 16])
                plsc.store_scatter(slab, [iota, col], lo)
                plsc.store_scatter(slab, [iota + 16, col], hi)
                nxt = i + _NBUF

                @pl.when(nxt < _B_PER_W)
                def _():
                    fetch(nxt, b)

            return 0

        lax.fori_loop(0, _B_PER_W // _NBUF, body, 0)
        pltpu.sync_copy(slab, out_hbm.at[:, pl.ds(base, _B_PER_W)])

    do_lookup(a_hbm, out_a)
    do_lookup(p_hbm, out_p)
    do_lookup(n_hbm, out_n)


@jax.jit
def kernel(anchor, positive, negative, W):
    mesh = plsc.VectorSubcoreMesh(core_axis_name="c", subcore_axis_name="s")
    out_t = jax.ShapeDtypeStruct((EMBED_DIM, BATCH), jnp.float32)
    f = pl.kernel(
        _triplet_gather,
        mesh=mesh,
        out_type=(out_t, out_t, out_t),
        scratch_types=[
            pltpu.VMEM((_B_PER_W,), jnp.int32),
            pltpu.VMEM((_NBUF, 8, EMBED_DIM), jnp.float32),
            pltpu.VMEM((EMBED_DIM, _B_PER_W), jnp.float32),
            pltpu.SemaphoreType.DMA((_NBUF,)),
        ],
        compiler_params=pltpu.CompilerParams(needs_layout_passes=False),
    )
    oa, op_, on = f(anchor, positive, negative,
                    jnp.reshape(W, (VOCAB // 8, 8, EMBED_DIM)))
    return (oa.T, op_.T, on.T)
